# Initial kernel scaffold; baseline (speedup 1.0000x reference)
import jax
import jax.numpy as jnp
from jax.experimental import pallas as pl


def kernel(node_feats, node_attrs, edge_embedding, edge_attrs, edge_index, W1, Wm1, Wm2, W2, Wu, W3, Wsc):
    raise NotImplementedError("write your pallas kernel here")


# speedup vs baseline: 2.1549x; 2.1549x over previous
import functools
import math

import jax
import jax.numpy as jnp
from jax import lax
from jax.experimental import pallas as pl
from jax.experimental.pallas import tpu as pltpu
from jax.experimental.pallas import tpu_sc as plsc

N = 10000
E = 320000
D = 128
DA = 16
DE = 16
H = 8
AVG_NEIGH = 32.0

NPAD = 10240
CHUNK = 512
NCHUNKS = E // CHUNK
STREAM = 128
NSTREAM = CHUNK // STREAM

_INV_SQRT_D = 1.0 / math.sqrt(float(D))
_INV_SQRT_DA = 1.0 / math.sqrt(float(DA))
_INV_SQRT_DE = 1.0 / math.sqrt(float(DE))
_INV_SQRT_H = 1.0 / math.sqrt(float(H))
_INV_SQRT_AVG = 1.0 / math.sqrt(AVG_NEIGH)
_INV_SQRT_DDA = 1.0 / math.sqrt(float(D * DA))


@functools.cache
def _sc_mesh2():
    return plsc.VectorSubcoreMesh(core_axis_name="c", subcore_axis_name="s")


@functools.cache
def _sc_mesh1():
    return plsc.VectorSubcoreMesh(
        core_axis_name="c", subcore_axis_name="s", num_cores=1
    )


def _x_body(nf_ref, w1_ref, x_ref):
    x_ref[...] = (
        jnp.dot(nf_ref[...], w1_ref[...], preferred_element_type=jnp.float32)
        * _INV_SQRT_D
    )


def _compute_x(node_feats, W1):
    bn = 1000
    return pl.pallas_call(
        _x_body,
        grid=(N // bn,),
        in_specs=[
            pl.BlockSpec((bn, D), lambda i: (i, 0)),
            pl.BlockSpec((D, D), lambda i: (0, 0)),
        ],
        out_specs=pl.BlockSpec((bn, D), lambda i: (i, 0)),
        out_shape=jax.ShapeDtypeStruct((N, D), jnp.float32),
    )(node_feats, W1)


def _gather_body(x_hbm, src_hbm, xs_hbm, idx0, idx1, idx2, idx3, rows_v, sem):
    c = lax.axis_index("c")
    s = lax.axis_index("s")
    wid = s * 2 + c
    idx = [idx0, idx1, idx2, idx3]

    def step(g, carry):
        cid = g * 32 + wid

        @pl.when(cid < NCHUNKS)
        def _():
            base = cid * CHUNK
            for j in range(NSTREAM):
                pltpu.sync_copy(
                    src_hbm.at[pl.ds(base + j * STREAM, STREAM)], idx[j]
                )
            copies = [
                pltpu.async_copy(
                    x_hbm.at[idx[j]],
                    rows_v.at[pl.ds(j * STREAM, STREAM)],
                    sem,
                )
                for j in range(NSTREAM)
            ]
            for cp in copies:
                cp.wait()
            pltpu.sync_copy(rows_v, xs_hbm.at[pl.ds(base, CHUNK)])

        return carry

    lax.fori_loop(0, (NCHUNKS + 31) // 32, step, 0)


def _gather_rows(x, src):
    fn = pl.kernel(
        _gather_body,
        out_type=jax.ShapeDtypeStruct((E, D), jnp.float32),
        mesh=_sc_mesh2(),
        scratch_types=[
            pltpu.VMEM((STREAM,), jnp.int32),
            pltpu.VMEM((STREAM,), jnp.int32),
            pltpu.VMEM((STREAM,), jnp.int32),
            pltpu.VMEM((STREAM,), jnp.int32),
            pltpu.VMEM((CHUNK, D), jnp.float32),
            pltpu.SemaphoreType.DMA,
        ],
    )
    return fn(x, src)


def _edge_body(ee_ref, ea_ref, xs_ref, wm1_ref, wm2_ref, w2_ref, msg_ref):
    h = jax.nn.silu(
        jnp.dot(ee_ref[...], wm1_ref[...], preferred_element_type=jnp.float32)
        * _INV_SQRT_DE
    )
    we = (
        jnp.dot(h, wm2_ref[...], preferred_element_type=jnp.float32)
        * _INV_SQRT_H
    )
    y = xs_ref[...] * ea_ref[...] * we
    z = (
        jnp.dot(y, w2_ref[...], preferred_element_type=jnp.float32)
        * _INV_SQRT_D
    )
    msg_ref[...] = jax.nn.silu(z)


def _edge_pipeline(edge_embedding, edge_attrs, xs, Wm1, Wm2, W2):
    be = 2560
    return pl.pallas_call(
        _edge_body,
        grid=(E // be,),
        in_specs=[
            pl.BlockSpec((be, DE), lambda i: (i, 0)),
            pl.BlockSpec((be, 1), lambda i: (i, 0)),
            pl.BlockSpec((be, D), lambda i: (i, 0)),
            pl.BlockSpec((DE, H), lambda i: (0, 0)),
            pl.BlockSpec((H, D), lambda i: (0, 0)),
            pl.BlockSpec((D, D), lambda i: (0, 0)),
        ],
        out_specs=pl.BlockSpec((be, D), lambda i: (i, 0)),
        out_shape=jax.ShapeDtypeStruct((E, D), jnp.float32),
    )(edge_embedding, edge_attrs, xs, Wm1, Wm2, W2)


NHALF = NPAD // 2
ACC_ROWS = NHALF + 128


def _scatter_body(msg_hbm, dst_hbm, zeros_hbm, out_hbm,
                  idx0, idx1, idx2, idx3, rows_v, zbuf, acc, sem):
    c = lax.axis_index("c")
    s = lax.axis_index("s")
    idx = [idx0, idx1, idx2, idx3]
    lo = c * NHALF

    pltpu.sync_copy(zeros_hbm, zbuf)
    for k in range(3):
        ch = s + k * 16

        @pl.when(ch < ACC_ROWS // 128)
        def _():
            pltpu.sync_copy(zbuf, acc.at[pl.ds(ch * 128, 128)])

    plsc.subcore_barrier()

    lane = lax.iota(jnp.int32, 16)

    def step(g, carry):
        cid = g * 16 + s

        @pl.when(cid < NCHUNKS)
        def _():
            base = cid * CHUNK
            for j in range(NSTREAM):
                pltpu.sync_copy(
                    dst_hbm.at[pl.ds(base + j * STREAM, STREAM)], idx[j]
                )
            pltpu.sync_copy(msg_hbm.at[pl.ds(base, CHUNK)], rows_v)
            for j in range(NSTREAM):
                for i in range(STREAM // 16):
                    t = idx[j][pl.ds(i * 16, 16)]
                    in_range = (t >= lo) & (t < lo + NHALF)
                    trash = NHALF + i * 16 + lane
                    idx[j][pl.ds(i * 16, 16)] = jnp.where(
                        in_range, t - lo, trash
                    )
            for j in range(NSTREAM):
                pltpu.sync_copy(
                    rows_v.at[pl.ds(j * STREAM, STREAM)],
                    acc.at[idx[j]],
                    add=True,
                )

        return carry

    lax.fori_loop(0, (NCHUNKS + 15) // 16, step, 0)
    plsc.subcore_barrier()

    for k in range(5):
        r0 = s * (NHALF // 16) + k * 64
        pltpu.sync_copy(acc.at[pl.ds(r0, 64)], zbuf.at[pl.ds(0, 64)])
        pltpu.sync_copy(zbuf.at[pl.ds(0, 64)], out_hbm.at[pl.ds(lo + r0, 64)])


def _scatter_rows(msg, dst, zeros):
    fn = pl.kernel(
        _scatter_body,
        out_type=jax.ShapeDtypeStruct((NPAD, D), jnp.float32),
        mesh=_sc_mesh2(),
        scratch_types=[
            pltpu.VMEM((STREAM,), jnp.int32),
            pltpu.VMEM((STREAM,), jnp.int32),
            pltpu.VMEM((STREAM,), jnp.int32),
            pltpu.VMEM((STREAM,), jnp.int32),
            pltpu.VMEM((CHUNK, D), jnp.float32),
            pltpu.VMEM((128, D), jnp.float32),
            pltpu.VMEM_SHARED((ACC_ROWS, D), jnp.float32),
            pltpu.SemaphoreType.DMA,
        ],
    )
    return fn(msg, dst, zeros)


def _finish_body(agg_ref, na_ref, nf_ref, wut_ref, w3_ref, wsc_ref, out_ref):
    agg = agg_ref[...] * _INV_SQRT_AVG
    u = (
        jnp.dot(na_ref[...], wut_ref[...], preferred_element_type=jnp.float32)
        * _INV_SQRT_DA
    )
    upd = (
        jnp.dot(agg * u, w3_ref[...], preferred_element_type=jnp.float32)
        * _INV_SQRT_D
    )
    na = na_ref[...]
    nf = nf_ref[...]
    sc_acc = jnp.zeros_like(upd)
    for v in range(DA):
        sc_acc = sc_acc + na[:, v : v + 1] * jnp.dot(
            nf, wsc_ref[:, v, :], preferred_element_type=jnp.float32
        )
    out_ref[...] = jax.nn.silu(upd + sc_acc * _INV_SQRT_DDA)


def _finish(agg, node_attrs, node_feats, WuT, W3, Wsc):
    bn = 1000
    return pl.pallas_call(
        _finish_body,
        grid=(N // bn,),
        in_specs=[
            pl.BlockSpec((bn, D), lambda i: (i, 0)),
            pl.BlockSpec((bn, DA), lambda i: (i, 0)),
            pl.BlockSpec((bn, D), lambda i: (i, 0)),
            pl.BlockSpec((DA, D), lambda i: (0, 0)),
            pl.BlockSpec((D, D), lambda i: (0, 0)),
            pl.BlockSpec((D, DA, D), lambda i: (0, 0, 0)),
        ],
        out_specs=pl.BlockSpec((bn, D), lambda i: (i, 0)),
        out_shape=jax.ShapeDtypeStruct((N, D), jnp.float32),
    )(agg, node_attrs, node_feats, WuT, W3, Wsc)


def kernel(node_feats, node_attrs, edge_embedding, edge_attrs, edge_index,
           W1, Wm1, Wm2, W2, Wu, W3, Wsc):
    src = edge_index[0]
    dst = edge_index[1]
    zeros = jnp.zeros((128, D), jnp.float32)

    x = _compute_x(node_feats, W1)
    xs = _gather_rows(x, src)
    msg = _edge_pipeline(edge_embedding, edge_attrs, xs, Wm1, Wm2, W2)
    agg = _scatter_rows(msg, dst, zeros)
    out = _finish(agg, node_attrs, node_feats, Wu.T, W3, Wsc)
    return out


# speedup vs baseline: 2.3779x; 1.1035x over previous
import functools
import math

import jax
import jax.numpy as jnp
from jax import lax
from jax.experimental import pallas as pl
from jax.experimental.pallas import tpu as pltpu
from jax.experimental.pallas import tpu_sc as plsc

N = 10000
E = 320000
D = 128
DA = 16
DE = 16
H = 8
AVG_NEIGH = 32.0

NPAD = 10240
CHUNK = 512
NCHUNKS = E // CHUNK
STREAM = 128
NSTREAM = CHUNK // STREAM

_INV_SQRT_D = 1.0 / math.sqrt(float(D))
_INV_SQRT_DA = 1.0 / math.sqrt(float(DA))
_INV_SQRT_DE = 1.0 / math.sqrt(float(DE))
_INV_SQRT_H = 1.0 / math.sqrt(float(H))
_INV_SQRT_AVG = 1.0 / math.sqrt(AVG_NEIGH)
_INV_SQRT_DDA = 1.0 / math.sqrt(float(D * DA))


@functools.cache
def _sc_mesh2():
    return plsc.VectorSubcoreMesh(core_axis_name="c", subcore_axis_name="s")


@functools.cache
def _sc_mesh1():
    return plsc.VectorSubcoreMesh(
        core_axis_name="c", subcore_axis_name="s", num_cores=1
    )


def _x_body(nf_ref, w1_ref, x_ref):
    x_ref[...] = (
        jnp.dot(nf_ref[...], w1_ref[...], preferred_element_type=jnp.float32)
        * _INV_SQRT_D
    )


def _compute_x(node_feats, W1):
    bn = 1000
    return pl.pallas_call(
        _x_body,
        grid=(N // bn,),
        in_specs=[
            pl.BlockSpec((bn, D), lambda i: (i, 0)),
            pl.BlockSpec((D, D), lambda i: (0, 0)),
        ],
        out_specs=pl.BlockSpec((bn, D), lambda i: (i, 0)),
        out_shape=jax.ShapeDtypeStruct((N, D), jnp.float32),
    )(node_feats, W1)


def _gather_body(x_hbm, src_hbm, xs_hbm, idx_v, rows_v, sem):
    c = lax.axis_index("c")
    s = lax.axis_index("s")
    wid = s * 2 + c

    def step(g, carry):
        cid = g * 32 + wid

        @pl.when(cid < NCHUNKS)
        def _():
            base = cid * CHUNK
            pltpu.sync_copy(src_hbm.at[cid], idx_v)
            copies = [
                pltpu.async_copy(
                    x_hbm.at[idx_v.at[j]],
                    rows_v.at[pl.ds(j * STREAM, STREAM)],
                    sem,
                )
                for j in range(NSTREAM)
            ]
            for cp in copies:
                cp.wait()
            pltpu.sync_copy(rows_v, xs_hbm.at[pl.ds(base, CHUNK)])

        return carry

    lax.fori_loop(0, (NCHUNKS + 31) // 32, step, 0)


def _gather_rows(x, src2d):
    fn = pl.kernel(
        _gather_body,
        out_type=jax.ShapeDtypeStruct((E, D), jnp.float32),
        mesh=_sc_mesh2(),
        scratch_types=[
            pltpu.VMEM((NSTREAM, STREAM), jnp.int32),
            pltpu.VMEM((CHUNK, D), jnp.float32),
            pltpu.SemaphoreType.DMA,
        ],
    )
    return fn(x, src2d)


def _edge_body(ee_ref, ea_ref, xs_ref, wm1_ref, wm2_ref, w2_ref, msg_ref):
    h = jax.nn.silu(
        jnp.dot(ee_ref[...], wm1_ref[...], preferred_element_type=jnp.float32)
        * _INV_SQRT_DE
    )
    we = (
        jnp.dot(h, wm2_ref[...], preferred_element_type=jnp.float32)
        * _INV_SQRT_H
    )
    y = xs_ref[...] * ea_ref[...] * we
    z = (
        jnp.dot(
            y.astype(jnp.bfloat16),
            w2_ref[...].astype(jnp.bfloat16),
            preferred_element_type=jnp.float32,
        )
        * _INV_SQRT_D
    )
    msg_ref[...] = jax.nn.silu(z)


def _edge_pipeline(edge_embedding, edge_attrs, xs, Wm1, Wm2, W2):
    be = 2560
    return pl.pallas_call(
        _edge_body,
        grid=(E // be,),
        in_specs=[
            pl.BlockSpec((be, DE), lambda i: (i, 0)),
            pl.BlockSpec((be, 1), lambda i: (i, 0)),
            pl.BlockSpec((be, D), lambda i: (i, 0)),
            pl.BlockSpec((DE, H), lambda i: (0, 0)),
            pl.BlockSpec((H, D), lambda i: (0, 0)),
            pl.BlockSpec((D, D), lambda i: (0, 0)),
        ],
        out_specs=pl.BlockSpec((be, D), lambda i: (i, 0)),
        out_shape=jax.ShapeDtypeStruct((E, D), jnp.float32),
    )(edge_embedding, edge_attrs, xs, Wm1, Wm2, W2)


NHALF = NPAD // 2
ACC_ROWS = NHALF + 128


def _scatter_body(msg_hbm, dst_hbm, zeros_hbm, out_hbm,
                  idx_v, rows_v, zbuf, acc, sem):
    c = lax.axis_index("c")
    s = lax.axis_index("s")
    lo = c * NHALF

    pltpu.sync_copy(zeros_hbm, zbuf)
    for k in range(3):
        ch = s + k * 16

        @pl.when(ch < ACC_ROWS // 128)
        def _():
            pltpu.sync_copy(zbuf, acc.at[pl.ds(ch * 128, 128)])

    plsc.subcore_barrier()

    lane = lax.iota(jnp.int32, 16)

    def step(g, carry):
        cid = g * 16 + s

        @pl.when(cid < NCHUNKS)
        def _():
            base = cid * CHUNK
            cpi = pltpu.async_copy(dst_hbm.at[cid], idx_v, sem)
            cpm = pltpu.async_copy(msg_hbm.at[pl.ds(base, CHUNK)], rows_v, sem)
            cpi.wait()
            cpm.wait()
            for j in range(NSTREAM):
                for i in range(STREAM // 16):
                    t = idx_v[j, pl.ds(i * 16, 16)]
                    in_range = (t >= lo) & (t < lo + NHALF)
                    trash = NHALF + i * 16 + lane
                    idx_v[j, pl.ds(i * 16, 16)] = jnp.where(
                        in_range, t - lo, trash
                    )
            adds = [
                pltpu.async_copy(
                    rows_v.at[pl.ds(j * STREAM, STREAM)],
                    acc.at[idx_v.at[j]],
                    sem,
                    add=True,
                )
                for j in range(NSTREAM)
            ]
            for cp in adds:
                cp.wait()

        return carry

    lax.fori_loop(0, (NCHUNKS + 15) // 16, step, 0)
    plsc.subcore_barrier()

    for k in range(5):
        r0 = s * (NHALF // 16) + k * 64
        pltpu.sync_copy(acc.at[pl.ds(r0, 64)], zbuf.at[pl.ds(0, 64)])
        pltpu.sync_copy(zbuf.at[pl.ds(0, 64)], out_hbm.at[pl.ds(lo + r0, 64)])


def _scatter_rows(msg, dst2d, zeros):
    fn = pl.kernel(
        _scatter_body,
        out_type=jax.ShapeDtypeStruct((NPAD, D), jnp.float32),
        mesh=_sc_mesh2(),
        scratch_types=[
            pltpu.VMEM((NSTREAM, STREAM), jnp.int32),
            pltpu.VMEM((CHUNK, D), jnp.float32),
            pltpu.VMEM((128, D), jnp.float32),
            pltpu.VMEM_SHARED((ACC_ROWS, D), jnp.float32),
            pltpu.SemaphoreType.DMA,
        ],
    )
    return fn(msg, dst2d, zeros)


def _finish_body(agg_ref, na_ref, nf_ref, wut_ref, w3_ref, wsc_ref, out_ref):
    agg = agg_ref[...] * _INV_SQRT_AVG
    u = (
        jnp.dot(na_ref[...], wut_ref[...], preferred_element_type=jnp.float32)
        * _INV_SQRT_DA
    )
    upd = (
        jnp.dot(agg * u, w3_ref[...], preferred_element_type=jnp.float32)
        * _INV_SQRT_D
    )
    na = na_ref[...]
    nf = nf_ref[...]
    sc_acc = jnp.zeros_like(upd)
    for v in range(DA):
        sc_acc = sc_acc + na[:, v : v + 1] * jnp.dot(
            nf, wsc_ref[:, v, :], preferred_element_type=jnp.float32
        )
    out_ref[...] = jax.nn.silu(upd + sc_acc * _INV_SQRT_DDA)


def _finish(agg, node_attrs, node_feats, WuT, W3, Wsc):
    bn = 1000
    return pl.pallas_call(
        _finish_body,
        grid=(N // bn,),
        in_specs=[
            pl.BlockSpec((bn, D), lambda i: (i, 0)),
            pl.BlockSpec((bn, DA), lambda i: (i, 0)),
            pl.BlockSpec((bn, D), lambda i: (i, 0)),
            pl.BlockSpec((DA, D), lambda i: (0, 0)),
            pl.BlockSpec((D, D), lambda i: (0, 0)),
            pl.BlockSpec((D, DA, D), lambda i: (0, 0, 0)),
        ],
        out_specs=pl.BlockSpec((bn, D), lambda i: (i, 0)),
        out_shape=jax.ShapeDtypeStruct((N, D), jnp.float32),
    )(agg, node_attrs, node_feats, WuT, W3, Wsc)


def kernel(node_feats, node_attrs, edge_embedding, edge_attrs, edge_index,
           W1, Wm1, Wm2, W2, Wu, W3, Wsc):
    src2d = edge_index[0].reshape(NCHUNKS, NSTREAM, STREAM)
    dst2d = edge_index[1].reshape(NCHUNKS, NSTREAM, STREAM)
    zeros = jnp.zeros((128, D), jnp.float32)

    x = _compute_x(node_feats, W1)
    xs = _gather_rows(x, src2d)
    msg = _edge_pipeline(edge_embedding, edge_attrs, xs, Wm1, Wm2, W2)
    agg = _scatter_rows(msg, dst2d, zeros)
    out = _finish(agg, node_attrs, node_feats, Wu.T, W3, Wsc)
    return out


# speedup vs baseline: 2.5473x; 1.0712x over previous
import functools
import math

import jax
import jax.numpy as jnp
from jax import lax
from jax.experimental import pallas as pl
from jax.experimental.pallas import tpu as pltpu
from jax.experimental.pallas import tpu_sc as plsc

N = 10000
E = 320000
D = 128
DA = 16
DE = 16
H = 8
AVG_NEIGH = 32.0

NPAD = 10240
CHUNK = 256
NCHUNKS = E // CHUNK
STREAM = 128
NSTREAM = CHUNK // STREAM

_INV_SQRT_D = 1.0 / math.sqrt(float(D))
_INV_SQRT_DA = 1.0 / math.sqrt(float(DA))
_INV_SQRT_DE = 1.0 / math.sqrt(float(DE))
_INV_SQRT_H = 1.0 / math.sqrt(float(H))
_INV_SQRT_AVG = 1.0 / math.sqrt(AVG_NEIGH)
_INV_SQRT_DDA = 1.0 / math.sqrt(float(D * DA))


@functools.cache
def _sc_mesh2():
    return plsc.VectorSubcoreMesh(core_axis_name="c", subcore_axis_name="s")


@functools.cache
def _sc_mesh1():
    return plsc.VectorSubcoreMesh(
        core_axis_name="c", subcore_axis_name="s", num_cores=1
    )


def _x_body(nf_ref, w1_ref, x_ref):
    x_ref[...] = (
        jnp.dot(nf_ref[...], w1_ref[...], preferred_element_type=jnp.float32)
        * _INV_SQRT_D
    )


def _compute_x(node_feats, W1):
    bn = 1000
    return pl.pallas_call(
        _x_body,
        grid=(N // bn,),
        in_specs=[
            pl.BlockSpec((bn, D), lambda i: (i, 0)),
            pl.BlockSpec((D, D), lambda i: (0, 0)),
        ],
        out_specs=pl.BlockSpec((bn, D), lambda i: (i, 0)),
        out_shape=jax.ShapeDtypeStruct((N, D), jnp.float32),
    )(node_feats, W1)


def _gather_body(x_hbm, src_hbm, xs_hbm, idxA, idxB, rowsA, rowsB,
                 si0, si1, sg0, sg1, sw0, sw1):
    c = lax.axis_index("c")
    s = lax.axis_index("s")
    wid = s * 2 + c
    idx = [idxA, idxB]
    rows = [rowsA, rowsB]
    semi = [si0, si1]
    semg = [sg0, sg1]
    semw = [sw0, sw1]
    T = (NCHUNKS + 31) // 32

    pltpu.async_copy(src_hbm.at[wid], idx[0], semi[0])

    def outer(t0, carry):
        for b in range(2):
            u = t0 * 2 + b
            nb = 1 - b
            cid = u * 32 + wid

            @pl.when((u >= 2) & ((u - 2) * 32 + wid < NCHUNKS))
            def _():
                pltpu.make_async_copy(
                    rows[b], xs_hbm.at[pl.ds(0, CHUNK)], semw[b]
                ).wait()

            @pl.when(cid < NCHUNKS)
            def _():
                pltpu.make_async_copy(src_hbm.at[wid], idx[b], semi[b]).wait()
                gs = [
                    pltpu.async_copy(
                        x_hbm.at[idx[b].at[j]],
                        rows[b].at[pl.ds(j * STREAM, STREAM)],
                        semg[b],
                    )
                    for j in range(NSTREAM)
                ]

                @pl.when((u + 1) * 32 + wid < NCHUNKS)
                def _():
                    pltpu.async_copy(
                        src_hbm.at[(u + 1) * 32 + wid], idx[nb], semi[nb]
                    )

                for cp in gs:
                    cp.wait()
                pltpu.async_copy(
                    rows[b], xs_hbm.at[pl.ds(cid * CHUNK, CHUNK)], semw[b]
                )

        return carry

    lax.fori_loop(0, T // 2, outer, 0)
    for b in range(2):
        u = T - 2 + b

        @pl.when(u * 32 + wid < NCHUNKS)
        def _(b=b):
            pltpu.make_async_copy(
                rows[b], xs_hbm.at[pl.ds(0, CHUNK)], semw[b]
            ).wait()


def _gather_rows(x, src3d):
    fn = pl.kernel(
        _gather_body,
        out_type=jax.ShapeDtypeStruct((E, D), jnp.float32),
        mesh=_sc_mesh2(),
        scratch_types=[
            pltpu.VMEM((NSTREAM, STREAM), jnp.int32),
            pltpu.VMEM((NSTREAM, STREAM), jnp.int32),
            pltpu.VMEM((CHUNK, D), jnp.float32),
            pltpu.VMEM((CHUNK, D), jnp.float32),
            pltpu.SemaphoreType.DMA,
            pltpu.SemaphoreType.DMA,
            pltpu.SemaphoreType.DMA,
            pltpu.SemaphoreType.DMA,
            pltpu.SemaphoreType.DMA,
            pltpu.SemaphoreType.DMA,
        ],
    )
    return fn(x, src3d)


def _edge_body(ee_ref, ea_ref, xs_ref, wm1_ref, wm2_ref, w2_ref, msg_ref):
    h = jax.nn.silu(
        jnp.dot(ee_ref[...], wm1_ref[...], preferred_element_type=jnp.float32)
        * _INV_SQRT_DE
    )
    we = (
        jnp.dot(h, wm2_ref[...], preferred_element_type=jnp.float32)
        * _INV_SQRT_H
    )
    y = xs_ref[...] * ea_ref[...] * we
    z = (
        jnp.dot(
            y.astype(jnp.bfloat16),
            w2_ref[...].astype(jnp.bfloat16),
            preferred_element_type=jnp.float32,
        )
        * _INV_SQRT_D
    )
    msg_ref[...] = jax.nn.silu(z)


def _edge_pipeline(edge_embedding, edge_attrs, xs, Wm1, Wm2, W2):
    be = 2560
    return pl.pallas_call(
        _edge_body,
        grid=(E // be,),
        in_specs=[
            pl.BlockSpec((be, DE), lambda i: (i, 0)),
            pl.BlockSpec((be, 1), lambda i: (i, 0)),
            pl.BlockSpec((be, D), lambda i: (i, 0)),
            pl.BlockSpec((DE, H), lambda i: (0, 0)),
            pl.BlockSpec((H, D), lambda i: (0, 0)),
            pl.BlockSpec((D, D), lambda i: (0, 0)),
        ],
        out_specs=pl.BlockSpec((be, D), lambda i: (i, 0)),
        out_shape=jax.ShapeDtypeStruct((E, D), jnp.float32),
    )(edge_embedding, edge_attrs, xs, Wm1, Wm2, W2)


NHALF = NPAD // 2
ACC_ROWS = NHALF + 128


def _scatter_body(msg_hbm, dst_hbm, zeros_hbm, out_hbm,
                  idxA, idxB, rowsA, rowsB, zbuf, acc,
                  sl0, sl1, sa0, sa1):
    c = lax.axis_index("c")
    s = lax.axis_index("s")
    lo = c * NHALF
    idx = [idxA, idxB]
    rows = [rowsA, rowsB]
    seml = [sl0, sl1]
    sema = [sa0, sa1]

    pltpu.sync_copy(zeros_hbm, zbuf)
    for k in range(3):
        ch = s + k * 16

        @pl.when(ch < ACC_ROWS // 128)
        def _():
            pltpu.sync_copy(zbuf, acc.at[pl.ds(ch * 128, 128)])

    plsc.subcore_barrier()

    lane = lax.iota(jnp.int32, 16)
    T = (NCHUNKS + 15) // 16 + 1

    pltpu.async_copy(dst_hbm.at[s], idx[0], seml[0])
    pltpu.async_copy(msg_hbm.at[pl.ds(s * CHUNK, CHUNK)], rows[0], seml[0])

    def outer(t0, carry):
        for b in range(2):
            u = t0 * 2 + b
            nb = 1 - b
            cid = u * 16 + s

            @pl.when(cid < NCHUNKS)
            def _():
                pltpu.make_async_copy(dst_hbm.at[s], idx[b], seml[b]).wait()
                pltpu.make_async_copy(
                    msg_hbm.at[pl.ds(0, CHUNK)], rows[b], seml[b]
                ).wait()
                for j in range(NSTREAM):
                    for i in range(STREAM // 16):
                        t = idx[b][j, pl.ds(i * 16, 16)]
                        in_range = (t >= lo) & (t < lo + NHALF)
                        trash = NHALF + i * 16 + lane
                        idx[b][j, pl.ds(i * 16, 16)] = jnp.where(
                            in_range, t - lo, trash
                        )
                for j in range(NSTREAM):
                    pltpu.async_copy(
                        rows[b].at[pl.ds(j * STREAM, STREAM)],
                        acc.at[idx[b].at[j]],
                        sema[b],
                        add=True,
                    )

            @pl.when((u >= 1) & ((u - 1) * 16 + s < NCHUNKS))
            def _():
                for j in range(NSTREAM):
                    pltpu.make_async_copy(
                        rows[nb].at[pl.ds(j * STREAM, STREAM)],
                        acc.at[pl.ds(0, STREAM)],
                        sema[nb],
                    ).wait()

            @pl.when((u + 1) * 16 + s < NCHUNKS)
            def _():
                pltpu.async_copy(
                    dst_hbm.at[(u + 1) * 16 + s], idx[nb], seml[nb]
                )
                pltpu.async_copy(
                    msg_hbm.at[pl.ds(((u + 1) * 16 + s) * CHUNK, CHUNK)],
                    rows[nb],
                    seml[nb],
                )

        return carry

    lax.fori_loop(0, T // 2, outer, 0)
    plsc.subcore_barrier()

    for k in range(5):
        r0 = s * (NHALF // 16) + k * 64
        pltpu.sync_copy(acc.at[pl.ds(r0, 64)], zbuf.at[pl.ds(0, 64)])
        pltpu.sync_copy(zbuf.at[pl.ds(0, 64)], out_hbm.at[pl.ds(lo + r0, 64)])


def _scatter_rows(msg, dst3d, zeros):
    fn = pl.kernel(
        _scatter_body,
        out_type=jax.ShapeDtypeStruct((NPAD, D), jnp.float32),
        mesh=_sc_mesh2(),
        scratch_types=[
            pltpu.VMEM((NSTREAM, STREAM), jnp.int32),
            pltpu.VMEM((NSTREAM, STREAM), jnp.int32),
            pltpu.VMEM((CHUNK, D), jnp.float32),
            pltpu.VMEM((CHUNK, D), jnp.float32),
            pltpu.VMEM((128, D), jnp.float32),
            pltpu.VMEM_SHARED((ACC_ROWS, D), jnp.float32),
            pltpu.SemaphoreType.DMA,
            pltpu.SemaphoreType.DMA,
            pltpu.SemaphoreType.DMA,
            pltpu.SemaphoreType.DMA,
        ],
    )
    return fn(msg, dst3d, zeros)


def _finish_body(agg_ref, na_ref, nf_ref, wut_ref, w3_ref, wsc_ref, out_ref):
    agg = agg_ref[...] * _INV_SQRT_AVG
    u = (
        jnp.dot(na_ref[...], wut_ref[...], preferred_element_type=jnp.float32)
        * _INV_SQRT_DA
    )
    upd = (
        jnp.dot(agg * u, w3_ref[...], preferred_element_type=jnp.float32)
        * _INV_SQRT_D
    )
    na = na_ref[...]
    nf = nf_ref[...]
    sc_acc = jnp.zeros_like(upd)
    for v in range(DA):
        sc_acc = sc_acc + na[:, v : v + 1] * jnp.dot(
            nf, wsc_ref[:, v, :], preferred_element_type=jnp.float32
        )
    out_ref[...] = jax.nn.silu(upd + sc_acc * _INV_SQRT_DDA)


def _finish(agg, node_attrs, node_feats, WuT, W3, Wsc):
    bn = 1000
    return pl.pallas_call(
        _finish_body,
        grid=(N // bn,),
        in_specs=[
            pl.BlockSpec((bn, D), lambda i: (i, 0)),
            pl.BlockSpec((bn, DA), lambda i: (i, 0)),
            pl.BlockSpec((bn, D), lambda i: (i, 0)),
            pl.BlockSpec((DA, D), lambda i: (0, 0)),
            pl.BlockSpec((D, D), lambda i: (0, 0)),
            pl.BlockSpec((D, DA, D), lambda i: (0, 0, 0)),
        ],
        out_specs=pl.BlockSpec((bn, D), lambda i: (i, 0)),
        out_shape=jax.ShapeDtypeStruct((N, D), jnp.float32),
    )(agg, node_attrs, node_feats, WuT, W3, Wsc)


def kernel(node_feats, node_attrs, edge_embedding, edge_attrs, edge_index,
           W1, Wm1, Wm2, W2, Wu, W3, Wsc):
    src3d = edge_index[0].reshape(NCHUNKS, NSTREAM, STREAM)
    dst3d = edge_index[1].reshape(NCHUNKS, NSTREAM, STREAM)
    zeros = jnp.zeros((128, D), jnp.float32)

    x = _compute_x(node_feats, W1)
    xs = _gather_rows(x, src3d)
    msg = _edge_pipeline(edge_embedding, edge_attrs, xs, Wm1, Wm2, W2)
    agg = _scatter_rows(msg, dst3d, zeros)
    out = _finish(agg, node_attrs, node_feats, Wu.T, W3, Wsc)
    return out


# speedup vs baseline: 2.6678x; 1.0473x over previous
import functools
import math

import jax
import jax.numpy as jnp
from jax import lax
from jax.experimental import pallas as pl
from jax.experimental.pallas import tpu as pltpu
from jax.experimental.pallas import tpu_sc as plsc

N = 10000
E = 320000
D = 128
DA = 16
DE = 16
H = 8
AVG_NEIGH = 32.0

NPAD = 10240
CHUNK = 256
NCHUNKS = E // CHUNK
STREAM = 128
NSTREAM = CHUNK // STREAM

_INV_SQRT_D = 1.0 / math.sqrt(float(D))
_INV_SQRT_DA = 1.0 / math.sqrt(float(DA))
_INV_SQRT_DE = 1.0 / math.sqrt(float(DE))
_INV_SQRT_H = 1.0 / math.sqrt(float(H))
_INV_SQRT_AVG = 1.0 / math.sqrt(AVG_NEIGH)
_INV_SQRT_DDA = 1.0 / math.sqrt(float(D * DA))


@functools.cache
def _sc_mesh2():
    return plsc.VectorSubcoreMesh(core_axis_name="c", subcore_axis_name="s")


@functools.cache
def _sc_mesh1():
    return plsc.VectorSubcoreMesh(
        core_axis_name="c", subcore_axis_name="s", num_cores=1
    )


def _x_body(nf_ref, w1_ref, x_ref):
    x_ref[...] = (
        jnp.dot(nf_ref[...], w1_ref[...], preferred_element_type=jnp.float32)
        * _INV_SQRT_D
    )


def _compute_x(node_feats, W1):
    bn = 1000
    return pl.pallas_call(
        _x_body,
        grid=(N // bn,),
        in_specs=[
            pl.BlockSpec((bn, D), lambda i: (i, 0)),
            pl.BlockSpec((D, D), lambda i: (0, 0)),
        ],
        out_specs=pl.BlockSpec((bn, D), lambda i: (i, 0)),
        out_shape=jax.ShapeDtypeStruct((N, D), jnp.float32),
    )(node_feats, W1)


def _gather_body(x_hbm, src_hbm, xs_hbm, idxA, idxB, rowsA, rowsB,
                 si0, si1, sg0, sg1, sw0, sw1):
    c = lax.axis_index("c")
    s = lax.axis_index("s")
    wid = s * 2 + c
    idx = [idxA, idxB]
    rows = [rowsA, rowsB]
    semi = [si0, si1]
    semg = [sg0, sg1]
    semw = [sw0, sw1]
    T = (NCHUNKS + 31) // 32

    pltpu.async_copy(src_hbm.at[wid], idx[0], semi[0])

    def outer(t0, carry):
        for b in range(2):
            u = t0 * 2 + b
            nb = 1 - b
            cid = u * 32 + wid

            @pl.when((u >= 2) & ((u - 2) * 32 + wid < NCHUNKS))
            def _():
                pltpu.make_async_copy(
                    rows[b], xs_hbm.at[pl.ds(0, CHUNK)], semw[b]
                ).wait()

            @pl.when(cid < NCHUNKS)
            def _():
                pltpu.make_async_copy(src_hbm.at[wid], idx[b], semi[b]).wait()
                gs = [
                    pltpu.async_copy(
                        x_hbm.at[idx[b].at[j]],
                        rows[b].at[pl.ds(j * STREAM, STREAM)],
                        semg[b],
                    )
                    for j in range(NSTREAM)
                ]

                @pl.when((u + 1) * 32 + wid < NCHUNKS)
                def _():
                    pltpu.async_copy(
                        src_hbm.at[(u + 1) * 32 + wid], idx[nb], semi[nb]
                    )

                for cp in gs:
                    cp.wait()
                pltpu.async_copy(
                    rows[b], xs_hbm.at[pl.ds(cid * CHUNK, CHUNK)], semw[b]
                )

        return carry

    lax.fori_loop(0, T // 2, outer, 0)
    for b in range(2):
        u = T - 2 + b

        @pl.when(u * 32 + wid < NCHUNKS)
        def _(b=b):
            pltpu.make_async_copy(
                rows[b], xs_hbm.at[pl.ds(0, CHUNK)], semw[b]
            ).wait()


def _gather_rows(x, src3d):
    fn = pl.kernel(
        _gather_body,
        out_type=jax.ShapeDtypeStruct((E, D), jnp.float32),
        mesh=_sc_mesh2(),
        scratch_types=[
            pltpu.VMEM((NSTREAM, STREAM), jnp.int32),
            pltpu.VMEM((NSTREAM, STREAM), jnp.int32),
            pltpu.VMEM((CHUNK, D), jnp.float32),
            pltpu.VMEM((CHUNK, D), jnp.float32),
            pltpu.SemaphoreType.DMA,
            pltpu.SemaphoreType.DMA,
            pltpu.SemaphoreType.DMA,
            pltpu.SemaphoreType.DMA,
            pltpu.SemaphoreType.DMA,
            pltpu.SemaphoreType.DMA,
        ],
    )
    return fn(x, src3d)


def _edge_body(ee_ref, ea_ref, xs_ref, wm1_ref, wm2_ref, w2_ref, msg_ref):
    h = jax.nn.silu(
        jnp.dot(ee_ref[...], wm1_ref[...], preferred_element_type=jnp.float32)
        * _INV_SQRT_DE
    )
    we = (
        jnp.dot(h, wm2_ref[...], preferred_element_type=jnp.float32)
        * _INV_SQRT_H
    )
    y = xs_ref[...] * ea_ref[...] * we
    z = (
        jnp.dot(
            y.astype(jnp.bfloat16),
            w2_ref[...].astype(jnp.bfloat16),
            preferred_element_type=jnp.float32,
        )
        * _INV_SQRT_D
    )
    msg_ref[...] = jax.nn.silu(z)


def _edge_pipeline(edge_embedding, edge_attrs, xs, Wm1, Wm2, W2):
    be = 8000
    return pl.pallas_call(
        _edge_body,
        grid=(E // be,),
        in_specs=[
            pl.BlockSpec((be, DE), lambda i: (i, 0)),
            pl.BlockSpec((be, 1), lambda i: (i, 0)),
            pl.BlockSpec((be, D), lambda i: (i, 0)),
            pl.BlockSpec((DE, H), lambda i: (0, 0)),
            pl.BlockSpec((H, D), lambda i: (0, 0)),
            pl.BlockSpec((D, D), lambda i: (0, 0)),
        ],
        out_specs=pl.BlockSpec((be, D), lambda i: (i, 0)),
        out_shape=jax.ShapeDtypeStruct((E, D), jnp.float32),
    )(edge_embedding, edge_attrs, xs, Wm1, Wm2, W2)


NHALF = NPAD // 2
ACC_ROWS = NHALF + 128


def _scatter_body(msg_hbm, dst_hbm, zeros_hbm, out_hbm,
                  idxA, idxB, rowsA, rowsB, zbuf, acc,
                  sl0, sl1, sa0, sa1):
    c = lax.axis_index("c")
    s = lax.axis_index("s")
    lo = c * NHALF
    idx = [idxA, idxB]
    rows = [rowsA, rowsB]
    seml = [sl0, sl1]
    sema = [sa0, sa1]

    pltpu.sync_copy(zeros_hbm, zbuf)
    for k in range(3):
        ch = s + k * 16

        @pl.when(ch < ACC_ROWS // 128)
        def _():
            pltpu.sync_copy(zbuf, acc.at[pl.ds(ch * 128, 128)])

    plsc.subcore_barrier()

    lane = lax.iota(jnp.int32, 16)
    T = (NCHUNKS + 15) // 16 + 1

    pltpu.async_copy(dst_hbm.at[s], idx[0], seml[0])
    pltpu.async_copy(msg_hbm.at[pl.ds(s * CHUNK, CHUNK)], rows[0], seml[0])

    def outer(t0, carry):
        for b in range(2):
            u = t0 * 2 + b
            nb = 1 - b
            cid = u * 16 + s

            @pl.when(cid < NCHUNKS)
            def _():
                pltpu.make_async_copy(dst_hbm.at[s], idx[b], seml[b]).wait()
                pltpu.make_async_copy(
                    msg_hbm.at[pl.ds(0, CHUNK)], rows[b], seml[b]
                ).wait()
                for j in range(NSTREAM):
                    for i in range(STREAM // 16):
                        t = idx[b][j, pl.ds(i * 16, 16)]
                        in_range = (t >= lo) & (t < lo + NHALF)
                        trash = NHALF + i * 16 + lane
                        idx[b][j, pl.ds(i * 16, 16)] = jnp.where(
                            in_range, t - lo, trash
                        )
                for j in range(NSTREAM):
                    pltpu.async_copy(
                        rows[b].at[pl.ds(j * STREAM, STREAM)],
                        acc.at[idx[b].at[j]],
                        sema[b],
                        add=True,
                    )

            @pl.when((u >= 1) & ((u - 1) * 16 + s < NCHUNKS))
            def _():
                for j in range(NSTREAM):
                    pltpu.make_async_copy(
                        rows[nb].at[pl.ds(j * STREAM, STREAM)],
                        acc.at[pl.ds(0, STREAM)],
                        sema[nb],
                    ).wait()

            @pl.when((u + 1) * 16 + s < NCHUNKS)
            def _():
                pltpu.async_copy(
                    dst_hbm.at[(u + 1) * 16 + s], idx[nb], seml[nb]
                )
                pltpu.async_copy(
                    msg_hbm.at[pl.ds(((u + 1) * 16 + s) * CHUNK, CHUNK)],
                    rows[nb],
                    seml[nb],
                )

        return carry

    lax.fori_loop(0, T // 2, outer, 0)
    plsc.subcore_barrier()

    for k in range(5):
        r0 = s * (NHALF // 16) + k * 64
        pltpu.sync_copy(acc.at[pl.ds(r0, 64)], zbuf.at[pl.ds(0, 64)])
        pltpu.sync_copy(zbuf.at[pl.ds(0, 64)], out_hbm.at[pl.ds(lo + r0, 64)])


def _scatter_rows(msg, dst3d, zeros):
    fn = pl.kernel(
        _scatter_body,
        out_type=jax.ShapeDtypeStruct((NPAD, D), jnp.float32),
        mesh=_sc_mesh2(),
        scratch_types=[
            pltpu.VMEM((NSTREAM, STREAM), jnp.int32),
            pltpu.VMEM((NSTREAM, STREAM), jnp.int32),
            pltpu.VMEM((CHUNK, D), jnp.float32),
            pltpu.VMEM((CHUNK, D), jnp.float32),
            pltpu.VMEM((128, D), jnp.float32),
            pltpu.VMEM_SHARED((ACC_ROWS, D), jnp.float32),
            pltpu.SemaphoreType.DMA,
            pltpu.SemaphoreType.DMA,
            pltpu.SemaphoreType.DMA,
            pltpu.SemaphoreType.DMA,
        ],
    )
    return fn(msg, dst3d, zeros)


def _finish_body(agg_ref, na_ref, nf_ref, wut_ref, w3_ref, wsc_ref, out_ref):
    agg = agg_ref[...] * _INV_SQRT_AVG
    u = (
        jnp.dot(na_ref[...], wut_ref[...], preferred_element_type=jnp.float32)
        * _INV_SQRT_DA
    )
    upd = (
        jnp.dot(agg * u, w3_ref[...], preferred_element_type=jnp.float32)
        * _INV_SQRT_D
    )
    na = na_ref[...]
    nf = nf_ref[...]
    sc_acc = jnp.zeros_like(upd)
    for v in range(DA):
        sc_acc = sc_acc + na[:, v : v + 1] * jnp.dot(
            nf, wsc_ref[:, v, :], preferred_element_type=jnp.float32
        )
    out_ref[...] = jax.nn.silu(upd + sc_acc * _INV_SQRT_DDA)


def _finish(agg, node_attrs, node_feats, WuT, W3, Wsc):
    bn = 1000
    return pl.pallas_call(
        _finish_body,
        grid=(N // bn,),
        in_specs=[
            pl.BlockSpec((bn, D), lambda i: (i, 0)),
            pl.BlockSpec((bn, DA), lambda i: (i, 0)),
            pl.BlockSpec((bn, D), lambda i: (i, 0)),
            pl.BlockSpec((DA, D), lambda i: (0, 0)),
            pl.BlockSpec((D, D), lambda i: (0, 0)),
            pl.BlockSpec((D, DA, D), lambda i: (0, 0, 0)),
        ],
        out_specs=pl.BlockSpec((bn, D), lambda i: (i, 0)),
        out_shape=jax.ShapeDtypeStruct((N, D), jnp.float32),
    )(agg, node_attrs, node_feats, WuT, W3, Wsc)


def kernel(node_feats, node_attrs, edge_embedding, edge_attrs, edge_index,
           W1, Wm1, Wm2, W2, Wu, W3, Wsc):
    src3d = edge_index[0].reshape(NCHUNKS, NSTREAM, STREAM)
    dst3d = edge_index[1].reshape(NCHUNKS, NSTREAM, STREAM)
    zeros = jnp.zeros((128, D), jnp.float32)

    x = _compute_x(node_feats, W1)
    xs = _gather_rows(x, src3d)
    msg = _edge_pipeline(edge_embedding, edge_attrs, xs, Wm1, Wm2, W2)
    agg = _scatter_rows(msg, dst3d, zeros)
    out = _finish(agg, node_attrs, node_feats, Wu.T, W3, Wsc)
    return out
